# Initial kernel scaffold; baseline (speedup 1.0000x reference)
import jax
import jax.numpy as jnp
from jax.experimental import pallas as pl


def kernel(features, W_heat, b_heat, W_mask, b_mask, W_part, b_part, W_kpts, b_kpts, W_mesh, b_mesh, v_template, shapedirs, posedirs, J_regressor, lbs_weights):
    raise NotImplementedError("write your pallas kernel here")


# speedup vs baseline: 1.1264x; 1.1264x over previous
import functools
import math

import jax
import jax.numpy as jnp
from jax import lax
from jax.experimental import pallas as pl
from jax.experimental.pallas import tpu as pltpu
from jax.experimental.pallas import tpu_sc as plsc

_C = 480
_CP = 512
_NK = 17
_NP = 15
_H = 128
_W = 128
_HW = _H * _W
_K = 30
_KP = 32
_NV = 6890
_NJ = 24
_VB = 1152
_T1 = 1024
_FOCAL = 5000.0
_CAM3D = 2.5
_PARENTS = (-1, 0, 0, 0, 1, 2, 3, 4, 5, 6, 7, 8, 9, 9, 9, 12, 13, 14, 16,
            17, 18, 19, 20, 21)
_F32 = jnp.float32
_I32 = jnp.int32


def _k1_body(f_ref, w_ref, b_ref, center_ref, mask_ref, part_ref, pk_ref,
             ft_ref):
    i = pl.program_id(0)
    f = f_ref[...]
    out = jnp.dot(w_ref[...], f, preferred_element_type=_F32) + b_ref[...]
    center_ref[...] = jnp.clip(jax.nn.sigmoid(out[0:1]), 1e-4, 1.0 - 1e-4)
    mask_ref[...] = jax.nn.sigmoid(out[1:2])
    part_ref[...] = out[2:17]
    off = out[17:51]
    flat = i * _T1 + lax.broadcasted_iota(_I32, (1, _T1), 1)
    px = (flat % _W).astype(_F32)
    py = (flat // _W).astype(_F32)
    row = lax.broadcasted_iota(_I32, (34, 1), 0)
    loc = jnp.where(row % 2 == 0, px, py)
    pk_ref[...] = loc - off
    ft_ref[...] = jnp.pad(f.T, ((0, 0), (0, _CP - _C)))


def _run_k1(F2, W64, b64):
    n = _HW // _T1
    return pl.pallas_call(
        _k1_body,
        grid=(n,),
        in_specs=[
            pl.BlockSpec((_C, _T1), lambda i: (0, i)),
            pl.BlockSpec((64, _C), lambda i: (0, 0)),
            pl.BlockSpec((64, 1), lambda i: (0, 0)),
        ],
        out_specs=[
            pl.BlockSpec((1, _T1), lambda i: (0, i)),
            pl.BlockSpec((1, _T1), lambda i: (0, i)),
            pl.BlockSpec((_NP, _T1), lambda i: (0, i)),
            pl.BlockSpec((34, _T1), lambda i: (0, i)),
            pl.BlockSpec((_T1, _CP), lambda i: (i, 0)),
        ],
        out_shape=[
            jax.ShapeDtypeStruct((1, _HW), _F32),
            jax.ShapeDtypeStruct((1, _HW), _F32),
            jax.ShapeDtypeStruct((_NP, _HW), _F32),
            jax.ShapeDtypeStruct((34, _HW), _F32),
            jax.ShapeDtypeStruct((_HW, _CP), _F32),
        ],
    )(F2, W64, b64)


def _sh_rows(a, off, fill):
    pad = jnp.full((1, a.shape[1]), fill, a.dtype)
    return (jnp.concatenate([a[1:], pad], axis=0) if off == 1
            else jnp.concatenate([pad, a[:-1]], axis=0))


def _sh_cols(a, off, fill):
    pad = jnp.full((a.shape[0], 1), fill, a.dtype)
    return (jnp.concatenate([a[:, 1:], pad], axis=1) if off == 1
            else jnp.concatenate([pad, a[:, :-1]], axis=1))


def _k2_body(c_ref, sc_ref, pos_ref, coord_ref, loc_ref):
    c = c_ref[...]
    rs = _sh_rows(c, 1, 0.0) + c + _sh_rows(c, -1, 0.0)
    ssum = _sh_cols(rs, 1, 0.0) + rs + _sh_cols(rs, -1, 0.0)
    c2 = (c + ssum / 9.0) * 0.5
    ninf = -jnp.inf
    rm = jnp.maximum(jnp.maximum(_sh_rows(c2, 1, ninf), c2),
                     _sh_rows(c2, -1, ninf))
    m3 = jnp.maximum(jnp.maximum(_sh_cols(rm, 1, ninf), rm),
                     _sh_cols(rm, -1, ninf))
    s = jnp.where(m3 == c2, c2, 0.0)

    ri = (lax.broadcasted_iota(_I32, (_H, _W), 0) * _W
          + lax.broadcasted_iota(_I32, (_H, _W), 1))
    lane = lax.broadcasted_iota(_I32, (1, _KP), 1)
    acc_v = jnp.zeros((1, _KP), _F32)
    acc_i = jnp.zeros((1, _KP), _I32)
    for t in range(_K):
        v = jnp.max(s)
        idx = jnp.min(jnp.where(s == v, ri, jnp.int32(1 << 30)))
        acc_v = jnp.where(lane == t, v, acc_v)
        acc_i = jnp.where(lane == t, idx, acc_i)
        s = jnp.where(ri == idx, jnp.float32(-1.0), s)
    y = acc_i // _W
    x = acc_i % _W
    sc_ref[...] = acc_v
    pos_ref[...] = acc_i
    coord_ref[...] = jnp.concatenate([y, x], axis=0)
    loc_ref[...] = jnp.concatenate([x, y], axis=0).astype(_F32) * 4.0


def _run_k2(center_img):
    return pl.pallas_call(
        _k2_body,
        out_shape=[
            jax.ShapeDtypeStruct((1, _KP), _F32),
            jax.ShapeDtypeStruct((1, _KP), _I32),
            jax.ShapeDtypeStruct((2, _KP), _I32),
            jax.ShapeDtypeStruct((2, _KP), _F32),
        ],
    )(center_img)


def _sc_gather(featT, idx32):
    mesh = plsc.VectorSubcoreMesh(core_axis_name="c", subcore_axis_name="s")

    @functools.partial(
        pl.kernel,
        mesh=mesh,
        out_type=jax.ShapeDtypeStruct((_KP, _CP), _F32),
        scratch_types=[
            pltpu.VMEM((_KP,), _I32),
            pltpu.VMEM((_KP, _CP), _F32),
            pltpu.SemaphoreType.DMA,
        ],
    )
    def k(ft_hbm, idx_hbm, out_hbm, idx_v, rows_v, sem):
        cid = lax.axis_index("c")
        sid = lax.axis_index("s")
        wid = sid * 2 + cid

        @pl.when(wid == 0)
        def _():
            pltpu.sync_copy(idx_hbm, idx_v)
            pltpu.async_copy(ft_hbm.at[idx_v], rows_v, sem).wait()
            pltpu.sync_copy(rows_v, out_hbm)

    return k(featT, idx32)


def _k4a_body(ip_ref, w_ref, b_ref, jb_ref, c11_ref, pf_ref, a_ref, cam_ref):
    par = (jnp.dot(ip_ref[...], w_ref[...], preferred_element_type=_F32)
           + b_ref[...])
    comp = [par[:, k * _NJ:(k + 1) * _NJ] for k in range(6)]
    a1 = comp[0:3]
    a2 = comp[3:6]
    n1 = jnp.sqrt(a1[0] * a1[0] + a1[1] * a1[1] + a1[2] * a1[2])
    b1 = [a1[c] / (n1 + 1e-8) for c in range(3)]
    d = b1[0] * a2[0] + b1[1] * a2[1] + b1[2] * a2[2]
    u = [a2[c] - d * b1[c] for c in range(3)]
    n2 = jnp.sqrt(u[0] * u[0] + u[1] * u[1] + u[2] * u[2])
    b2 = [u[c] / (n2 + 1e-8) for c in range(3)]
    b3 = [b1[1] * b2[2] - b1[2] * b2[1],
          b1[2] * b2[0] - b1[0] * b2[2],
          b1[0] * b2[1] - b1[1] * b2[0]]
    cols = [b1, b2, b3]
    r = [[cols[n][m] for n in range(3)] for m in range(3)]

    ones = jnp.ones((_KP, 1), _F32)
    c11 = jnp.concatenate([ones, par[:, 144:154]], axis=1)
    pf = jnp.concatenate(
        [r[m][n][:, 1:] - (1.0 if m == n else 0.0)
         for m in range(3) for n in range(3)], axis=1)

    Jc = [jnp.dot(c11, jb_ref[c], preferred_element_type=_F32)
          for c in range(3)]

    def rj(j):
        return [[r[m][n][:, j:j + 1] for n in range(3)] for m in range(3)]

    Rg = [None] * _NJ
    tg = [None] * _NJ
    Rg[0] = rj(0)
    tg[0] = [Jc[c][:, 0:1] for c in range(3)]
    for j in range(1, _NJ):
        p = _PARENTS[j]
        L = rj(j)
        tl = [Jc[c][:, j:j + 1] - Jc[c][:, p:p + 1] for c in range(3)]
        Rg[j] = [[Rg[p][m][0] * L[0][n] + Rg[p][m][1] * L[1][n]
                  + Rg[p][m][2] * L[2][n] for n in range(3)]
                 for m in range(3)]
        tg[j] = [Rg[p][m][0] * tl[0] + Rg[p][m][1] * tl[1]
                 + Rg[p][m][2] * tl[2] + tg[p][m] for m in range(3)]

    blocks = []
    for m in range(3):
        for n in range(4):
            if n < 3:
                cols_j = [Rg[j][m][n] for j in range(_NJ)]
            else:
                cols_j = [tg[j][m]
                          - (Rg[j][m][0] * Jc[0][:, j:j + 1]
                             + Rg[j][m][1] * Jc[1][:, j:j + 1]
                             + Rg[j][m][2] * Jc[2][:, j:j + 1])
                          for j in range(_NJ)]
            blocks.append(jnp.concatenate(cols_j, axis=1))
    a_ref[...] = jnp.concatenate(blocks, axis=0)

    kv = math.sqrt(_FOCAL * _FOCAL * _CAM3D * _CAM3D / (512.0 * 512.0))
    tz = kv * jnp.exp(par[:, 156:157] * math.log(1.1))
    cam_ref[...] = jnp.concatenate(
        [par[:, 154:155], par[:, 155:156], tz, jnp.zeros((_KP, 1), _F32)],
        axis=1)
    c11_ref[...] = c11
    pf_ref[...] = pf


def _run_k4a(ip, Wmp, bp, Jb):
    return pl.pallas_call(
        _k4a_body,
        out_shape=[
            jax.ShapeDtypeStruct((_KP, 11), _F32),
            jax.ShapeDtypeStruct((_KP, 207), _F32),
            jax.ShapeDtypeStruct((384, _NJ), _F32),
            jax.ShapeDtypeStruct((_KP, 4), _F32),
        ],
    )(ip, Wmp, bp, Jb)


def _k4b_body(c11_ref, pf_ref, a_ref, cam_ref, dsv_ref, pd_ref, lw_ref,
              out_ref):
    tn = (((1,), (1,)), ((), ()))
    Tall = lax.dot_general(a_ref[...], lw_ref[...], tn,
                           preferred_element_type=_F32)
    pf = pf_ref[...]
    c11 = c11_ref[...]
    vp = []
    for c in range(3):
        v1 = lax.dot_general(pf, pd_ref[:, c, :], tn,
                             preferred_element_type=_F32)
        v0 = jnp.dot(c11, dsv_ref[:, c, :], preferred_element_type=_F32)
        vp.append(v0 + v1)
    for m in range(3):
        acc = Tall[(m * 4 + 3) * _KP:(m * 4 + 4) * _KP]
        for n in range(3):
            acc = acc + Tall[(m * 4 + n) * _KP:(m * 4 + n + 1) * _KP] * vp[n]
        out_ref[:, m, :] = acc + cam_ref[:, m:m + 1]


def _run_k4b(c11, pf, A, cam, Dsv, posedirs, lbs_weights):
    n = -(-_NV // _VB)
    return pl.pallas_call(
        _k4b_body,
        grid=(n,),
        in_specs=[
            pl.BlockSpec((_KP, 11), lambda i: (0, 0)),
            pl.BlockSpec((_KP, 207), lambda i: (0, 0)),
            pl.BlockSpec((384, _NJ), lambda i: (0, 0)),
            pl.BlockSpec((_KP, 4), lambda i: (0, 0)),
            pl.BlockSpec((11, 3, _VB), lambda i: (0, 0, i)),
            pl.BlockSpec((_VB, 3, 207), lambda i: (i, 0, 0)),
            pl.BlockSpec((_VB, _NJ), lambda i: (i, 0)),
        ],
        out_specs=pl.BlockSpec((_KP, 3, _VB), lambda i: (0, 0, i)),
        out_shape=jax.ShapeDtypeStruct((_KP, 3, _NV), _F32),
    )(c11, pf, A, cam, Dsv, posedirs, lbs_weights)


def kernel(features, W_heat, b_heat, W_mask, b_mask, W_part, b_part, W_kpts,
           b_kpts, W_mesh, b_mesh, v_template, shapedirs, posedirs,
           J_regressor, lbs_weights):
    F2 = features.reshape(_C, _HW)
    W51 = jnp.concatenate([W_heat[-1:], W_mask, W_part, W_kpts], axis=0)
    W64 = jnp.pad(W51, ((0, 13), (0, 0)))
    b51 = jnp.concatenate([b_heat[-1:], b_mask, b_part, b_kpts], axis=0)
    b64 = jnp.pad(b51, (0, 13)).reshape(64, 1)

    center, mask2, part2, pk2, featT = _run_k1(F2, W64, b64)
    sc2, pos2, coord2, loc2 = _run_k2(center.reshape(_H, _W))
    ip32 = _sc_gather(featT, pos2.reshape(_KP))[:, :_C]

    perm = ([j * 6 + c for c in range(6) for j in range(_NJ)]
            + list(range(144, 157)))
    Wmp = jnp.pad(W_mesh[jnp.array(perm)].T, ((0, 0), (0, 3)))
    bp = jnp.pad(b_mesh[jnp.array(perm)], (0, 3)).reshape(1, 160)
    Jb = jnp.concatenate(
        [(J_regressor @ v_template).T[:, None, :],
         jnp.einsum('jv,vcl->clj', J_regressor, shapedirs)], axis=1)
    Dsv = jnp.concatenate(
        [v_template.T[None], shapedirs.transpose(2, 1, 0)], axis=0)
    pperm = jnp.array([(j - 1) * 9 + c for c in range(9)
                       for j in range(1, _NJ)])
    pd = jnp.take(posedirs, pperm, axis=2)

    c11, pf, A, cam = _run_k4a(ip32, Wmp, bp, Jb)
    meshp = _run_k4b(c11, pf, A, cam, Dsv, pd, lbs_weights)

    instance_coord = coord2.T[:_K]
    instance_imgid = jnp.zeros((_K,), _I32)
    instance_param = ip32[:_K]
    scores = sc2.reshape(_KP)[:_K]
    mesh = meshp[:_K].transpose(0, 2, 1)
    location = loc2.T[:_K]
    pred_keypoints = pk2.reshape(1, 34, _H, _W)
    mask = mask2.reshape(1, 1, _H, _W)
    part = part2.reshape(1, _NP, _H, _W)
    return (instance_coord, instance_imgid, instance_param, scores, mesh,
            location, pred_keypoints, mask, part)


# speedup vs baseline: 1.8633x; 1.6542x over previous
import math

import jax
import jax.numpy as jnp
from jax import lax
from jax.experimental import pallas as pl
from jax.experimental.pallas import tpu as pltpu

_C = 480
_CP = 512
_NK = 17
_NP = 15
_H = 128
_W = 128
_HW = _H * _W
_K = 30
_KP = 32
_NV = 6890
_NJ = 24
_VB = 1152
_T1 = 1024
_FOCAL = 5000.0
_CAM3D = 2.5
_PARENTS = (-1, 0, 0, 0, 1, 2, 3, 4, 5, 6, 7, 8, 9, 9, 9, 12, 13, 14, 16,
            17, 18, 19, 20, 21)
_F32 = jnp.float32
_I32 = jnp.int32


def _k1_body(f_ref, w_ref, b_ref, center_ref, mask_ref, part_ref, pk_ref,
             ft_ref):
    i = pl.program_id(0)
    f = f_ref[...]
    out = jnp.dot(w_ref[...], f, preferred_element_type=_F32) + b_ref[...]
    center_ref[...] = jnp.clip(jax.nn.sigmoid(out[0:1]), 1e-4, 1.0 - 1e-4)
    mask_ref[...] = jax.nn.sigmoid(out[1:2])
    part_ref[...] = out[2:17]
    off = out[17:51]
    flat = i * _T1 + lax.broadcasted_iota(_I32, (1, _T1), 1)
    px = (flat % _W).astype(_F32)
    py = (flat // _W).astype(_F32)
    row = lax.broadcasted_iota(_I32, (34, 1), 0)
    loc = jnp.where(row % 2 == 0, px, py)
    pk_ref[...] = loc - off
    ft_ref[...] = jnp.pad(f.T, ((0, 0), (0, _CP - _C)))


def _run_k1(F2, W64, b64):
    n = _HW // _T1
    return pl.pallas_call(
        _k1_body,
        grid=(n,),
        in_specs=[
            pl.BlockSpec((_C, _T1), lambda i: (0, i)),
            pl.BlockSpec((64, _C), lambda i: (0, 0)),
            pl.BlockSpec((64, 1), lambda i: (0, 0)),
        ],
        out_specs=[
            pl.BlockSpec((1, _T1), lambda i: (0, i)),
            pl.BlockSpec((1, _T1), lambda i: (0, i)),
            pl.BlockSpec((_NP, _T1), lambda i: (0, i)),
            pl.BlockSpec((34, _T1), lambda i: (0, i)),
            pl.BlockSpec((_T1, _CP), lambda i: (i, 0)),
        ],
        out_shape=[
            jax.ShapeDtypeStruct((1, _HW), _F32),
            jax.ShapeDtypeStruct((1, _HW), _F32),
            jax.ShapeDtypeStruct((_NP, _HW), _F32),
            jax.ShapeDtypeStruct((34, _HW), _F32),
            jax.ShapeDtypeStruct((_HW, _CP), _F32),
        ],
    )(F2, W64, b64)


def _sh_rows(a, off, fill):
    pad = jnp.full((1, a.shape[1]), fill, a.dtype)
    return (jnp.concatenate([a[1:], pad], axis=0) if off == 1
            else jnp.concatenate([pad, a[:-1]], axis=0))


def _sh_cols(a, off, fill):
    pad = jnp.full((a.shape[0], 1), fill, a.dtype)
    return (jnp.concatenate([a[:, 1:], pad], axis=1) if off == 1
            else jnp.concatenate([pad, a[:, :-1]], axis=1))


def _k2_body(c_ref, sc_ref, pos_ref, coord_ref, loc_ref):
    c = c_ref[...]
    rs = _sh_rows(c, 1, 0.0) + c + _sh_rows(c, -1, 0.0)
    ssum = _sh_cols(rs, 1, 0.0) + rs + _sh_cols(rs, -1, 0.0)
    c2 = (c + ssum / 9.0) * 0.5
    ninf = -jnp.inf
    rm = jnp.maximum(jnp.maximum(_sh_rows(c2, 1, ninf), c2),
                     _sh_rows(c2, -1, ninf))
    m3 = jnp.maximum(jnp.maximum(_sh_cols(rm, 1, ninf), rm),
                     _sh_cols(rm, -1, ninf))
    s = jnp.where(m3 == c2, c2, 0.0)

    ri = (lax.broadcasted_iota(_I32, (_H, _W), 0) * _W
          + lax.broadcasted_iota(_I32, (_H, _W), 1))
    lane = lax.broadcasted_iota(_I32, (1, _KP), 1)
    acc_v = jnp.zeros((1, _KP), _F32)
    acc_i = jnp.zeros((1, _KP), _I32)
    for t in range(_K):
        v = jnp.max(s)
        idx = jnp.min(jnp.where(s == v, ri, jnp.int32(1 << 30)))
        acc_v = jnp.where(lane == t, v, acc_v)
        acc_i = jnp.where(lane == t, idx, acc_i)
        s = jnp.where(ri == idx, jnp.float32(-1.0), s)
    y = acc_i // _W
    x = acc_i % _W
    sc_ref[...] = acc_v
    pos_ref[...] = acc_i
    coord_ref[...] = jnp.concatenate([y, x], axis=0)
    loc_ref[...] = jnp.concatenate([x, y], axis=0).astype(_F32) * 4.0


def _run_k2(center_img):
    return pl.pallas_call(
        _k2_body,
        out_shape=[
            jax.ShapeDtypeStruct((1, _KP), _F32),
            jax.ShapeDtypeStruct((1, _KP), _I32),
            jax.ShapeDtypeStruct((2, _KP), _I32),
            jax.ShapeDtypeStruct((2, _KP), _F32),
        ],
    )(center_img)


def _gather_body(idx_ref, ft_ref, out_ref, sem):
    copies = [
        pltpu.make_async_copy(
            ft_ref.at[pl.ds(idx_ref[p], 1), :],
            out_ref.at[pl.ds(p, 1), :],
            sem,
        )
        for p in range(_KP)
    ]
    for c in copies:
        c.start()
    for c in copies:
        c.wait()


def _sc_gather(featT, idx32):
    return pl.pallas_call(
        _gather_body,
        grid_spec=pltpu.PrefetchScalarGridSpec(
            num_scalar_prefetch=1,
            grid=(1,),
            in_specs=[pl.BlockSpec(memory_space=pl.ANY)],
            out_specs=pl.BlockSpec((_KP, _CP), lambda i, idx: (0, 0)),
            scratch_shapes=[pltpu.SemaphoreType.DMA],
        ),
        out_shape=jax.ShapeDtypeStruct((_KP, _CP), _F32),
    )(idx32, featT)


def _k4a_body(ip_ref, w_ref, b_ref, jb_ref, c11_ref, pf_ref, a_ref, cam_ref):
    par = (jnp.dot(ip_ref[...], w_ref[...], preferred_element_type=_F32)
           + b_ref[...])
    comp = [par[:, k * _NJ:(k + 1) * _NJ] for k in range(6)]
    a1 = comp[0:3]
    a2 = comp[3:6]
    n1 = jnp.sqrt(a1[0] * a1[0] + a1[1] * a1[1] + a1[2] * a1[2])
    b1 = [a1[c] / (n1 + 1e-8) for c in range(3)]
    d = b1[0] * a2[0] + b1[1] * a2[1] + b1[2] * a2[2]
    u = [a2[c] - d * b1[c] for c in range(3)]
    n2 = jnp.sqrt(u[0] * u[0] + u[1] * u[1] + u[2] * u[2])
    b2 = [u[c] / (n2 + 1e-8) for c in range(3)]
    b3 = [b1[1] * b2[2] - b1[2] * b2[1],
          b1[2] * b2[0] - b1[0] * b2[2],
          b1[0] * b2[1] - b1[1] * b2[0]]
    cols = [b1, b2, b3]
    r = [[cols[n][m] for n in range(3)] for m in range(3)]

    ones = jnp.ones((_KP, 1), _F32)
    c11 = jnp.concatenate([ones, par[:, 144:154]], axis=1)
    pf = jnp.concatenate(
        [r[m][n][:, j:j + 1] - (1.0 if m == n else 0.0)
         for j in range(1, _NJ) for m in range(3) for n in range(3)],
        axis=1)

    Jc = [jnp.dot(c11, jb_ref[c], preferred_element_type=_F32)
          for c in range(3)]

    def rj(j):
        return [[r[m][n][:, j:j + 1] for n in range(3)] for m in range(3)]

    Rg = [None] * _NJ
    tg = [None] * _NJ
    Rg[0] = rj(0)
    tg[0] = [Jc[c][:, 0:1] for c in range(3)]
    for j in range(1, _NJ):
        p = _PARENTS[j]
        L = rj(j)
        tl = [Jc[c][:, j:j + 1] - Jc[c][:, p:p + 1] for c in range(3)]
        Rg[j] = [[Rg[p][m][0] * L[0][n] + Rg[p][m][1] * L[1][n]
                  + Rg[p][m][2] * L[2][n] for n in range(3)]
                 for m in range(3)]
        tg[j] = [Rg[p][m][0] * tl[0] + Rg[p][m][1] * tl[1]
                 + Rg[p][m][2] * tl[2] + tg[p][m] for m in range(3)]

    blocks = []
    for m in range(3):
        for n in range(4):
            if n < 3:
                cols_j = [Rg[j][m][n] for j in range(_NJ)]
            else:
                cols_j = [tg[j][m]
                          - (Rg[j][m][0] * Jc[0][:, j:j + 1]
                             + Rg[j][m][1] * Jc[1][:, j:j + 1]
                             + Rg[j][m][2] * Jc[2][:, j:j + 1])
                          for j in range(_NJ)]
            blocks.append(jnp.concatenate(cols_j, axis=1))
    a_ref[...] = jnp.concatenate(blocks, axis=0)

    kv = math.sqrt(_FOCAL * _FOCAL * _CAM3D * _CAM3D / (512.0 * 512.0))
    tz = kv * jnp.exp(par[:, 156:157] * math.log(1.1))
    cam_ref[...] = jnp.concatenate(
        [par[:, 154:155], par[:, 155:156], tz, jnp.zeros((_KP, 1), _F32)],
        axis=1)
    c11_ref[...] = c11
    pf_ref[...] = pf


def _run_k4a(ip, Wmp, bp, Jb):
    return pl.pallas_call(
        _k4a_body,
        out_shape=[
            jax.ShapeDtypeStruct((_KP, 11), _F32),
            jax.ShapeDtypeStruct((_KP, 207), _F32),
            jax.ShapeDtypeStruct((384, _NJ), _F32),
            jax.ShapeDtypeStruct((_KP, 4), _F32),
        ],
    )(ip, Wmp, bp, Jb)


def _k4b_body(c11_ref, pf_ref, a_ref, cam_ref, dsv_ref, pd_ref, lw_ref,
              out_ref):
    tn = (((1,), (1,)), ((), ()))
    Tall = lax.dot_general(a_ref[...], lw_ref[...], tn,
                           preferred_element_type=_F32)
    pf = pf_ref[...]
    c11 = c11_ref[...]
    vp = []
    for c in range(3):
        v1 = lax.dot_general(pf, pd_ref[:, c, :], tn,
                             preferred_element_type=_F32)
        v0 = jnp.dot(c11, dsv_ref[:, c, :], preferred_element_type=_F32)
        vp.append(v0 + v1)
    for m in range(3):
        acc = Tall[(m * 4 + 3) * _KP:(m * 4 + 4) * _KP]
        for n in range(3):
            acc = acc + Tall[(m * 4 + n) * _KP:(m * 4 + n + 1) * _KP] * vp[n]
        out_ref[:, m, :] = acc + cam_ref[:, m:m + 1]


def _run_k4b(c11, pf, A, cam, Dsv, posedirs, lbs_weights):
    n = -(-_NV // _VB)
    return pl.pallas_call(
        _k4b_body,
        grid=(n,),
        in_specs=[
            pl.BlockSpec((_KP, 11), lambda i: (0, 0)),
            pl.BlockSpec((_KP, 207), lambda i: (0, 0)),
            pl.BlockSpec((384, _NJ), lambda i: (0, 0)),
            pl.BlockSpec((_KP, 4), lambda i: (0, 0)),
            pl.BlockSpec((11, 3, _VB), lambda i: (0, 0, i)),
            pl.BlockSpec((_VB, 3, 207), lambda i: (i, 0, 0)),
            pl.BlockSpec((_VB, _NJ), lambda i: (i, 0)),
        ],
        out_specs=pl.BlockSpec((_KP, 3, _VB), lambda i: (0, 0, i)),
        out_shape=jax.ShapeDtypeStruct((_KP, 3, _NV), _F32),
    )(c11, pf, A, cam, Dsv, posedirs, lbs_weights)


def kernel(features, W_heat, b_heat, W_mask, b_mask, W_part, b_part, W_kpts,
           b_kpts, W_mesh, b_mesh, v_template, shapedirs, posedirs,
           J_regressor, lbs_weights):
    F2 = features.reshape(_C, _HW)
    W51 = jnp.concatenate([W_heat[-1:], W_mask, W_part, W_kpts], axis=0)
    W64 = jnp.pad(W51, ((0, 13), (0, 0)))
    b51 = jnp.concatenate([b_heat[-1:], b_mask, b_part, b_kpts], axis=0)
    b64 = jnp.pad(b51, (0, 13)).reshape(64, 1)

    center, mask2, part2, pk2, featT = _run_k1(F2, W64, b64)
    sc2, pos2, coord2, loc2 = _run_k2(center.reshape(_H, _W))
    ip32 = _sc_gather(featT, pos2.reshape(_KP))[:, :_C]

    perm = ([j * 6 + c for c in range(6) for j in range(_NJ)]
            + list(range(144, 157)))
    Wmp = jnp.pad(W_mesh[jnp.array(perm)].T, ((0, 0), (0, 3)))
    bp = jnp.pad(b_mesh[jnp.array(perm)], (0, 3)).reshape(1, 160)
    Jb = jnp.concatenate(
        [(J_regressor @ v_template).T[:, None, :],
         jnp.einsum('jv,vcl->clj', J_regressor, shapedirs)], axis=1)
    Dsv = jnp.concatenate(
        [v_template.T[None], shapedirs.transpose(2, 1, 0)], axis=0)
    c11, pf, A, cam = _run_k4a(ip32, Wmp, bp, Jb)
    meshp = _run_k4b(c11, pf, A, cam, Dsv, posedirs, lbs_weights)

    instance_coord = coord2.T[:_K]
    instance_imgid = jnp.zeros((_K,), _I32)
    instance_param = ip32[:_K]
    scores = sc2.reshape(_KP)[:_K]
    mesh = meshp[:_K].transpose(0, 2, 1)
    location = loc2.T[:_K]
    pred_keypoints = pk2.reshape(1, 34, _H, _W)
    mask = mask2.reshape(1, 1, _H, _W)
    part = part2.reshape(1, _NP, _H, _W)
    return (instance_coord, instance_imgid, instance_param, scores, mesh,
            location, pred_keypoints, mask, part)
